# Initial kernel scaffold; baseline (speedup 1.0000x reference)
import jax
import jax.numpy as jnp
from jax.experimental import pallas as pl


def kernel(space_features, all_features, neighbors_matrix, num_entries, params):
    raise NotImplementedError("write your pallas kernel here")


# speedup vs baseline: 48.7895x; 48.7895x over previous
import functools

import jax
import jax.numpy as jnp
from jax import lax
from jax.experimental import pallas as pl
from jax.experimental.pallas import tpu as pltpu
from jax.experimental.pallas import tpu_sc as plsc

F32 = jnp.float32
I32 = jnp.int32
NW = 32
KP1 = 17


def _ceil16(x):
    return (x + 15) // 16 * 16


def _mm_body(x_ref, w_ref, o_ref):
    o_ref[...] = lax.dot(x_ref[...], w_ref[...],
                         precision=lax.Precision.HIGHEST,
                         preferred_element_type=F32)


def _tc_matmul(x, w, bm=512):
    m, p = x.shape
    n = w.shape[1]
    return pl.pallas_call(
        _mm_body,
        grid=(m // bm,),
        in_specs=[pl.BlockSpec((bm, p), lambda i: (i, 0)),
                  pl.BlockSpec((p, n), lambda i: (0, 0))],
        out_specs=pl.BlockSpec((bm, n), lambda i: (i, 0)),
        out_shape=jax.ShapeDtypeStruct((m, n), F32),
    )(x, w)


def _seg_list(total):
    segs = []
    off = 0
    while off < total:
        s = min(128, total - off)
        segs.append((off, s))
        off += s
    return segs


def _sc_gather_sum(ytab, idx, bias, nn, dpad, c):
    npw = nn // NW
    nchunks = npw // c
    segs = _seg_list(c * KP1)
    nslice = dpad // 16
    mesh = plsc.VectorSubcoreMesh(core_axis_name="c", subcore_axis_name="s")

    @functools.partial(
        pl.kernel, mesh=mesh,
        out_type=jax.ShapeDtypeStruct((nn, dpad), F32),
        compiler_params=pltpu.CompilerParams(use_tc_tiling_on_sc=False),
        scratch_types=[
            pltpu.VMEM((c * KP1,), I32),
            pltpu.VMEM((c * KP1, dpad), F32),
            pltpu.VMEM((c, dpad), F32),
            pltpu.VMEM((dpad,), F32),
            pltpu.SemaphoreType.DMA,
        ],
    )
    def k(y_hbm, idx_hbm, bias_hbm, out_hbm, idx_v, rows_v, out_v, bias_v,
          sem):
        wid = lax.axis_index("s") * 2 + lax.axis_index("c")
        base_node = wid * npw
        pltpu.sync_copy(bias_hbm, bias_v)

        def chunk_body(t, carry):
            node0 = base_node + t * c
            pltpu.sync_copy(idx_hbm.at[pl.ds(node0 * KP1, c * KP1)], idx_v)
            descs = []
            for off, seg in segs:
                descs.append(pltpu.async_copy(
                    y_hbm.at[idx_v.at[pl.ds(off, seg)]],
                    rows_v.at[pl.ds(off, seg)], sem))
            for d in descs:
                d.wait()

            def acc_body(i, carry2):
                r0 = i * KP1
                for s in range(nslice):
                    sl = pl.ds(16 * s, 16)
                    a = rows_v[r0, sl]
                    for kk in range(1, KP1):
                        a = a + rows_v[r0 + kk, sl]
                    out_v[i, sl] = jnp.maximum(a + bias_v[sl], 0.0)
                return carry2

            lax.fori_loop(0, c, acc_body, 0)
            pltpu.sync_copy(out_v, out_hbm.at[pl.ds(node0, c)])
            return carry

        lax.fori_loop(0, nchunks, chunk_body, 0)

    return k(ytab, idx, bias)


def _sc_masked_mean(z, n_arr, b, e, dpad, fdim):
    fpad = _ceil16(fdim)
    nsl = fpad // 16
    rows_chunk = 512
    nch = e // rows_chunk
    mesh = plsc.VectorSubcoreMesh(core_axis_name="c", subcore_axis_name="s")

    @functools.partial(
        pl.kernel, mesh=mesh,
        out_type=jax.ShapeDtypeStruct((b, fpad), F32),
        compiler_params=pltpu.CompilerParams(use_tc_tiling_on_sc=False),
        scratch_types=[
            pltpu.VMEM((rows_chunk, dpad), F32),
            pltpu.VMEM((16,), I32),
            pltpu.VMEM((fpad,), F32),
        ],
    )
    def k(z_hbm, n_hbm, out_hbm, zrows_v, n_v, out_v):
        wid = lax.axis_index("s") * 2 + lax.axis_index("c")
        pltpu.sync_copy(n_hbm.at[wid], n_v)
        nsplat = n_v[pl.ds(0, 16)]
        iota = lax.iota(I32, 16)
        accs = [jnp.zeros((16,), F32) for _ in range(nsl)]
        for ch in range(nch):
            pltpu.sync_copy(z_hbm.at[pl.ds(wid * e + ch * rows_chunk,
                                           rows_chunk)], zrows_v)

            def ebody(i, carry):
                pred = (ch * rows_chunk + i) < nsplat
                out = []
                for s in range(nsl):
                    lanes_valid = 16 * s + iota < fdim
                    v = jnp.where(pred & lanes_valid,
                                  zrows_v[i, pl.ds(16 * s, 16)], 0.0)
                    out.append(carry[s] + v)
                return tuple(out)

            accs = lax.fori_loop(0, rows_chunk, ebody, tuple(accs))
        inv = 1.0 / jnp.maximum(nsplat, 1).astype(F32)
        for s in range(nsl):
            out_v[pl.ds(16 * s, 16)] = accs[s] * inv
        pltpu.sync_copy(out_v, out_hbm.at[wid])

    return k(z, n_arr)


def _head_body(x_ref, w1_ref, b1_ref, w2_ref, b2_ref, w3_ref, b3_ref,
               lg_ref, pred_ref):
    x = x_ref[...]
    h = jnp.maximum(lax.dot(x, w1_ref[...], precision=lax.Precision.HIGHEST,
                            preferred_element_type=F32) + b1_ref[...], 0.0)
    h = jnp.maximum(lax.dot(h, w2_ref[...], precision=lax.Precision.HIGHEST,
                            preferred_element_type=F32) + b2_ref[...], 0.0)
    lg = lax.dot(h, w3_ref[...], precision=lax.Precision.HIGHEST,
                 preferred_element_type=F32) + b3_ref[...]
    lg_ref[...] = lg
    ncls = lg.shape[1]
    col = lax.broadcasted_iota(I32, lg.shape, 1)
    mx = jnp.max(lg, axis=1, keepdims=True)
    pred_ref[...] = jnp.min(jnp.where(lg >= mx, col, ncls), axis=1,
                            keepdims=True)


def _tc_head(flat, w1, b1, w2, b2, w3, b3):
    b = flat.shape[0]
    ncls = w3.shape[1]
    return pl.pallas_call(
        _head_body,
        out_shape=(jax.ShapeDtypeStruct((b, ncls), F32),
                   jax.ShapeDtypeStruct((b, 1), I32)),
    )(flat, w1, b1, w2, b2, w3, b3)


def _build_wbig(wa, ws, fa, fs, p, out):
    kk = wa.shape[0] // (fa + fs)
    dpad = _ceil16(2 * out)
    wa_r = wa.reshape(kk, fa + fs, out)
    ws_r = ws.reshape(kk, fa + fs, out)
    blocks = jnp.concatenate([wa_r, ws_r], axis=2)
    corr = -jnp.concatenate([wa_r[:, fa:, :].sum(0),
                             ws_r[:, fa:, :].sum(0)], axis=1)
    corr_full = jnp.zeros((fa + fs, 2 * out), F32).at[fa:].set(corr)
    wb = jnp.concatenate([blocks, corr_full[None]], axis=0)
    wb = jnp.pad(wb, ((0, 0), (0, p - (fa + fs)), (0, dpad - 2 * out)))
    return wb.transpose(1, 0, 2).reshape(p, KP1 * dpad)


def _chunk_nodes(dpad):
    for c in (64, 32, 16):
        if c * KP1 * dpad * 4 <= 220_000:
            return c
    return 16


def kernel(space_features, all_features, neighbors_matrix, num_entries,
           params):
    b, e, fs0 = space_features.shape
    fa0 = all_features.shape[2]
    kk = neighbors_matrix.shape[2]
    nn = b * e
    nlayers = 6
    layer_out = [params['W%da' % l].shape[1] for l in range(nlayers)]

    nbr = neighbors_matrix.astype(I32)
    bofs = (jnp.arange(b, dtype=I32) * e)[:, None, None]
    idx_nbr = (bofs + nbr) * KP1 + jnp.arange(kk, dtype=I32)[None, None, :]
    self_row = (bofs[..., 0] + jnp.arange(e, dtype=I32)[None, :]) * KP1 + kk
    idx = jnp.concatenate([idx_nbr, self_row[:, :, None]],
                          axis=2).reshape(-1)

    z = jnp.concatenate([all_features.reshape(nn, fa0),
                         space_features.reshape(nn, fs0)], axis=1)
    fa, fs = fa0, fs0
    for l in range(nlayers):
        out = layer_out[l]
        dpad = _ceil16(2 * out)
        p = z.shape[1]
        wbig = _build_wbig(params['W%da' % l], params['W%ds' % l],
                           fa, fs, p, out)
        bias = jnp.pad(jnp.concatenate([params['b%da' % l],
                                        params['b%ds' % l]]),
                       (0, dpad - 2 * out))
        y = _tc_matmul(z, wbig)
        ytab = y.reshape(nn * KP1, dpad)
        z = _sc_gather_sum(ytab, idx, bias, nn, dpad, _chunk_nodes(dpad))
        fa = fs = out

    n_rep = jnp.tile(num_entries.reshape(b, 1).astype(I32), (1, 16))
    flat = _sc_masked_mean(z, n_rep, b, e, z.shape[1], layer_out[-1])
    f1 = jnp.pad(params['fc1_w'], ((0, flat.shape[1] - layer_out[-1]),
                                   (0, 0)))
    logits, pred = _tc_head(flat, f1, params['fc1_b'][None],
                            params['fc2_w'], params['fc2_b'][None],
                            params['fc3_w'], params['fc3_b'][None])
    return logits, pred[:, 0]


# speedup vs baseline: 55.4074x; 1.1356x over previous
import functools

import jax
import jax.numpy as jnp
from jax import lax
from jax.experimental import pallas as pl
from jax.experimental.pallas import tpu as pltpu
from jax.experimental.pallas import tpu_sc as plsc

F32 = jnp.float32
I32 = jnp.int32
NW = 32
KP1 = 17


def _ceil16(x):
    return (x + 15) // 16 * 16


def _mm_body(x_ref, w_ref, o_ref):
    o_ref[...] = lax.dot(x_ref[...], w_ref[...],
                         precision=lax.Precision.HIGHEST,
                         preferred_element_type=F32)


def _tc_matmul(x, w, bm=512):
    m, p = x.shape
    n = w.shape[1]
    return pl.pallas_call(
        _mm_body,
        grid=(m // bm,),
        in_specs=[pl.BlockSpec((bm, p), lambda i: (i, 0)),
                  pl.BlockSpec((p, n), lambda i: (0, 0))],
        out_specs=pl.BlockSpec((bm, n), lambda i: (i, 0)),
        out_shape=jax.ShapeDtypeStruct((m, n), F32),
    )(x, w)


def _seg_list(total):
    segs = []
    off = 0
    while off < total:
        s = min(128, total - off)
        segs.append((off, s))
        off += s
    return segs


def _sc_gather_sum(ytab, idx, bias, nn, dpad, c):
    npw = nn // NW
    nchunks = npw // c
    segs = _seg_list(c * KP1)
    nslice = dpad // 16
    mesh = plsc.VectorSubcoreMesh(core_axis_name="c", subcore_axis_name="s")

    @functools.partial(
        pl.kernel, mesh=mesh,
        out_type=jax.ShapeDtypeStruct((nn, dpad), F32),
        compiler_params=pltpu.CompilerParams(use_tc_tiling_on_sc=False),
        scratch_types=[
            pltpu.VMEM((c * KP1,), I32), pltpu.VMEM((c * KP1,), I32),
            pltpu.VMEM((c * KP1, dpad), F32),
            pltpu.VMEM((c * KP1, dpad), F32),
            pltpu.VMEM((c, dpad), F32),
            pltpu.VMEM((dpad,), F32),
            pltpu.SemaphoreType.DMA, pltpu.SemaphoreType.DMA,
        ],
    )
    def k(y_hbm, idx_hbm, bias_hbm, out_hbm, idx_a, idx_b, rows_a, rows_b,
          out_v, bias_v, sem_a, sem_b):
        wid = lax.axis_index("s") * 2 + lax.axis_index("c")
        base_node = wid * npw
        pltpu.sync_copy(bias_hbm, bias_v)
        idx_bufs, row_bufs, sems = (idx_a, idx_b), (rows_a, rows_b), \
            (sem_a, sem_b)

        def fire(g, bi):
            node0 = base_node + g * c
            pltpu.sync_copy(idx_hbm.at[pl.ds(node0 * KP1, c * KP1)],
                            idx_bufs[bi])
            for off, seg in segs:
                pltpu.async_copy(y_hbm.at[idx_bufs[bi].at[pl.ds(off, seg)]],
                                 row_bufs[bi].at[pl.ds(off, seg)], sems[bi])

        def drain(bi):
            for off, seg in segs:
                pltpu.make_async_copy(
                    y_hbm.at[idx_bufs[bi].at[pl.ds(off, seg)]],
                    row_bufs[bi].at[pl.ds(off, seg)], sems[bi]).wait()

        def process(g, bi):
            rows_v = row_bufs[bi]

            def acc_body(i, carry2):
                r0 = i * KP1
                for s in range(nslice):
                    sl = pl.ds(16 * s, 16)
                    a = rows_v[r0, sl]
                    for kk in range(1, KP1):
                        a = a + rows_v[r0 + kk, sl]
                    out_v[i, sl] = jnp.maximum(a + bias_v[sl], 0.0)
                return carry2

            lax.fori_loop(0, c, acc_body, 0)
            pltpu.sync_copy(out_v, out_hbm.at[pl.ds(base_node + g * c, c)])

        fire(0, 0)

        def group(to, carry):
            for b in (0, 1):
                g = 2 * to + b

                @pl.when(g + 1 < nchunks)
                def _():
                    fire(g + 1, 1 - b)

                drain(b)
                process(g, b)
            return carry

        lax.fori_loop(0, nchunks // 2, group, 0)

    return k(ytab, idx, bias)


def _sc_masked_mean(z, n_arr, b, e, dpad, fdim):
    fpad = _ceil16(fdim)
    nsl = fpad // 16
    rows_chunk = 512
    nch = e // rows_chunk
    mesh = plsc.VectorSubcoreMesh(core_axis_name="c", subcore_axis_name="s")

    @functools.partial(
        pl.kernel, mesh=mesh,
        out_type=jax.ShapeDtypeStruct((b, fpad), F32),
        compiler_params=pltpu.CompilerParams(use_tc_tiling_on_sc=False),
        scratch_types=[
            pltpu.VMEM((rows_chunk, dpad), F32),
            pltpu.VMEM((16,), I32),
            pltpu.VMEM((fpad,), F32),
        ],
    )
    def k(z_hbm, n_hbm, out_hbm, zrows_v, n_v, out_v):
        wid = lax.axis_index("s") * 2 + lax.axis_index("c")
        pltpu.sync_copy(n_hbm.at[wid], n_v)
        nsplat = n_v[pl.ds(0, 16)]
        iota = lax.iota(I32, 16)
        accs = [jnp.zeros((16,), F32) for _ in range(nsl)]
        for ch in range(nch):
            pltpu.sync_copy(z_hbm.at[pl.ds(wid * e + ch * rows_chunk,
                                           rows_chunk)], zrows_v)

            def ebody(i, carry):
                pred = (ch * rows_chunk + i) < nsplat
                out = []
                for s in range(nsl):
                    lanes_valid = 16 * s + iota < fdim
                    v = jnp.where(pred & lanes_valid,
                                  zrows_v[i, pl.ds(16 * s, 16)], 0.0)
                    out.append(carry[s] + v)
                return tuple(out)

            accs = lax.fori_loop(0, rows_chunk, ebody, tuple(accs))
        inv = 1.0 / jnp.maximum(nsplat, 1).astype(F32)
        for s in range(nsl):
            out_v[pl.ds(16 * s, 16)] = accs[s] * inv
        pltpu.sync_copy(out_v, out_hbm.at[wid])

    return k(z, n_arr)


def _head_body(x_ref, w1_ref, b1_ref, w2_ref, b2_ref, w3_ref, b3_ref,
               lg_ref, pred_ref):
    x = x_ref[...]
    h = jnp.maximum(lax.dot(x, w1_ref[...], precision=lax.Precision.HIGHEST,
                            preferred_element_type=F32) + b1_ref[...], 0.0)
    h = jnp.maximum(lax.dot(h, w2_ref[...], precision=lax.Precision.HIGHEST,
                            preferred_element_type=F32) + b2_ref[...], 0.0)
    lg = lax.dot(h, w3_ref[...], precision=lax.Precision.HIGHEST,
                 preferred_element_type=F32) + b3_ref[...]
    lg_ref[...] = lg
    ncls = lg.shape[1]
    col = lax.broadcasted_iota(I32, lg.shape, 1)
    mx = jnp.max(lg, axis=1, keepdims=True)
    pred_ref[...] = jnp.min(jnp.where(lg >= mx, col, ncls), axis=1,
                            keepdims=True)


def _tc_head(flat, w1, b1, w2, b2, w3, b3):
    b = flat.shape[0]
    ncls = w3.shape[1]
    return pl.pallas_call(
        _head_body,
        out_shape=(jax.ShapeDtypeStruct((b, ncls), F32),
                   jax.ShapeDtypeStruct((b, 1), I32)),
    )(flat, w1, b1, w2, b2, w3, b3)


def _build_wbig(wa, ws, fa, fs, p, out):
    kk = wa.shape[0] // (fa + fs)
    dpad = _ceil16(2 * out)
    wa_r = wa.reshape(kk, fa + fs, out)
    ws_r = ws.reshape(kk, fa + fs, out)
    blocks = jnp.concatenate([wa_r, ws_r], axis=2)
    corr = -jnp.concatenate([wa_r[:, fa:, :].sum(0),
                             ws_r[:, fa:, :].sum(0)], axis=1)
    corr_full = jnp.zeros((fa + fs, 2 * out), F32).at[fa:].set(corr)
    wb = jnp.concatenate([blocks, corr_full[None]], axis=0)
    wb = jnp.pad(wb, ((0, 0), (0, p - (fa + fs)), (0, dpad - 2 * out)))
    return wb.transpose(1, 0, 2).reshape(p, KP1 * dpad)


def _chunk_nodes(dpad):
    for c in (64, 32, 16):
        if c * KP1 * dpad * 4 <= 220_000:
            return c
    return 16


def kernel(space_features, all_features, neighbors_matrix, num_entries,
           params):
    b, e, fs0 = space_features.shape
    fa0 = all_features.shape[2]
    kk = neighbors_matrix.shape[2]
    nn = b * e
    nlayers = 6
    layer_out = [params['W%da' % l].shape[1] for l in range(nlayers)]

    nbr = neighbors_matrix.astype(I32)
    bofs = (jnp.arange(b, dtype=I32) * e)[:, None, None]
    idx_nbr = (bofs + nbr) * KP1 + jnp.arange(kk, dtype=I32)[None, None, :]
    self_row = (bofs[..., 0] + jnp.arange(e, dtype=I32)[None, :]) * KP1 + kk
    idx = jnp.concatenate([idx_nbr, self_row[:, :, None]],
                          axis=2).reshape(-1)

    z = jnp.concatenate([all_features.reshape(nn, fa0),
                         space_features.reshape(nn, fs0)], axis=1)
    fa, fs = fa0, fs0
    for l in range(nlayers):
        out = layer_out[l]
        dpad = _ceil16(2 * out)
        p = z.shape[1]
        wbig = _build_wbig(params['W%da' % l], params['W%ds' % l],
                           fa, fs, p, out)
        bias = jnp.pad(jnp.concatenate([params['b%da' % l],
                                        params['b%ds' % l]]),
                       (0, dpad - 2 * out))
        y = _tc_matmul(z, wbig)
        ytab = y.reshape(nn * KP1, dpad)
        z = _sc_gather_sum(ytab, idx, bias, nn, dpad, _chunk_nodes(dpad))
        fa = fs = out

    n_rep = jnp.tile(num_entries.reshape(b, 1).astype(I32), (1, 16))
    flat = _sc_masked_mean(z, n_rep, b, e, z.shape[1], layer_out[-1])
    f1 = jnp.pad(params['fc1_w'], ((0, flat.shape[1] - layer_out[-1]),
                                   (0, 0)))
    logits, pred = _tc_head(flat, f1, params['fc1_b'][None],
                            params['fc2_w'], params['fc2_b'][None],
                            params['fc3_w'], params['fc3_b'][None])
    return logits, pred[:, 0]
